# baseline (speedup 1.0000x reference)
import jax
import jax.numpy as jnp
from jax.experimental import pallas as pl
from jax.experimental.pallas import tpu as pltpu

_K = 64


def _enc_body(x_ref, w_ref, b_ref, o_ref):
    acc = jax.lax.dot_general(
        x_ref[...], w_ref[...],
        (((1,), (1,)), ((), ())),
        preferred_element_type=jnp.float32,
    )
    o_ref[...] = jnp.maximum(acc + b_ref[...], 0.0)


def _encode(x_bf16, W_enc_bf16, b_enc, *, br=512, bd=1024):
    B, D_IN = x_bf16.shape
    D_DICT = W_enc_bf16.shape[0]
    grid = (D_DICT // bd, B // br)
    return pl.pallas_call(
        _enc_body,
        grid=grid,
        in_specs=[
            pl.BlockSpec((br, D_IN), lambda j, i: (i, 0)),
            pl.BlockSpec((bd, D_IN), lambda j, i: (j, 0)),
            pl.BlockSpec((1, bd), lambda j, i: (0, j)),
        ],
        out_specs=pl.BlockSpec((br, bd), lambda j, i: (i, j)),
        out_shape=jax.ShapeDtypeStruct((B, D_DICT), jnp.float32),
    )(x_bf16, W_enc_bf16, b_enc.reshape(1, D_DICT))


_J_MAX = 4


def _lane_count(m):
    w = m.shape[1]
    while w > 128:
        w //= 2
        m = jax.lax.slice(m, (0, 0), (m.shape[0], w)) + jax.lax.slice(
            m, (0, w), (m.shape[0], 2 * w))
    return jnp.sum(m.astype(jnp.float32), axis=1, keepdims=True)


def _topk_body(pre_ref, lat_ref, p16_ref):
    pre = pre_ref[...]
    bits = pltpu.bitcast(pre, jnp.int32)
    rows, _ = pre.shape

    p16_ref[...] = pltpu.bitcast(
        jnp.bitwise_and(bits, jnp.int32(-65536)), jnp.float32
    ).astype(jnp.bfloat16)

    def body16(_, carry):
        lo, hi, c_lo, c_hi = carry
        mid = lo + jax.lax.shift_right_logical(hi - lo, 1)
        t = pltpu.bitcast(mid << 16, jnp.float32).astype(jnp.bfloat16)
        mask = jnp.where(p16_ref[...] >= t, jnp.bfloat16(1), jnp.bfloat16(0))
        cnt = _lane_count(mask)
        ok = cnt >= float(_K)
        return (jnp.where(ok, mid, lo), jnp.where(ok, hi, mid),
                jnp.where(ok, cnt, c_lo), jnp.where(ok, c_hi, cnt))

    carry = (jnp.zeros((rows, 1), jnp.int32),
             jnp.full((rows, 1), 0x7F80, jnp.int32),
             jnp.full((rows, 1), float(pre.shape[1]), jnp.float32),
             jnp.zeros((rows, 1), jnp.float32))
    lo16, _, c_lo, c_hi = jax.lax.fori_loop(0, 15, body16, carry)

    def body32(_, carry):
        lo, hi, c_lo, c_hi = carry
        mid = lo + jax.lax.shift_right_logical(hi - lo, 1)
        t = pltpu.bitcast(mid, jnp.float32)
        cnt = _lane_count(jnp.where(pre >= t, 1.0, 0.0))
        ok = cnt >= float(_K)
        return (jnp.where(ok, mid, lo), jnp.where(ok, hi, mid),
                jnp.where(ok, cnt, c_lo), jnp.where(ok, c_hi, cnt))

    lo = lo16 << 16
    carry = (lo, lo + 0x10000, c_lo, c_hi)
    lo, hi, c_lo, c_hi = jax.lax.fori_loop(0, 4, body32, carry)

    bucket_lo = pltpu.bitcast(lo, jnp.float32)
    bucket_hi = pltpu.bitcast(hi, jnp.float32)
    r = float(_K) - c_hi

    z0 = jnp.where((pre >= bucket_lo) & (pre < bucket_hi), pre, -1.0)
    m_j = jnp.max(z0, axis=1, keepdims=True)
    sel = jnp.where(r == 1.0, m_j, -1.0)
    for j in range(2, _J_MAX + 1):
        m_j = jnp.max(jnp.where(z0 < m_j, z0, -1.0), axis=1, keepdims=True)
        sel = jnp.where(r == float(j), m_j, sel)
    t_val = jnp.where(c_lo <= float(_K), bucket_lo,
                      jnp.maximum(sel, bucket_lo))
    lat_ref[...] = jnp.where(pre >= t_val, pre, 0.0)


def _topk_mask(pre, *, br=128):
    B, D_DICT = pre.shape
    return pl.pallas_call(
        _topk_body,
        grid=(B // br,),
        in_specs=[pl.BlockSpec((br, D_DICT), lambda i: (i, 0))],
        out_specs=pl.BlockSpec((br, D_DICT), lambda i: (i, 0)),
        out_shape=jax.ShapeDtypeStruct((B, D_DICT), jnp.float32),
        scratch_shapes=[pltpu.VMEM((br, D_DICT), jnp.bfloat16)],
    )(pre)


def _dec_body(lat_ref, w_ref, b_ref, o_ref):
    k = pl.program_id(1)

    @pl.when(k == 0)
    def _():
        o_ref[...] = jnp.broadcast_to(b_ref[...], o_ref.shape)

    lat = lat_ref[...].astype(jnp.bfloat16)
    o_ref[...] += jax.lax.dot_general(
        lat, w_ref[...],
        (((1,), (1,)), ((), ())),
        preferred_element_type=jnp.float32,
    )


def _decode(latents, W_dec_bf16, b_dec, *, br=1024, bk=2048):
    B, D_DICT = latents.shape
    D_IN = W_dec_bf16.shape[0]
    grid = (B // br, D_DICT // bk)
    return pl.pallas_call(
        _dec_body,
        grid=grid,
        in_specs=[
            pl.BlockSpec((br, bk), lambda i, k: (i, k)),
            pl.BlockSpec((D_IN, bk), lambda i, k: (0, k)),
            pl.BlockSpec((1, D_IN), lambda i, k: (0, 0)),
        ],
        out_specs=pl.BlockSpec((br, D_IN), lambda i, k: (i, 0)),
        out_shape=jax.ShapeDtypeStruct((B, D_IN), jnp.float32),
    )(latents, W_dec_bf16, b_dec.reshape(1, D_IN))


def kernel(x, W_enc, b_enc, W_dec, b_dec):
    pre = _encode(x.astype(jnp.bfloat16), W_enc.astype(jnp.bfloat16), b_enc)
    latents = _topk_mask(pre)
    x_hat = _decode(latents, W_dec.astype(jnp.bfloat16), b_dec)
    return (x_hat, latents)


# speedup vs baseline: 1.0429x; 1.0429x over previous
import jax
import jax.numpy as jnp
from jax.experimental import pallas as pl
from jax.experimental.pallas import tpu as pltpu

_K = 64


def _enc_body(x_ref, w_ref, b_ref, o_ref):
    acc = jax.lax.dot_general(
        x_ref[...], w_ref[...],
        (((1,), (1,)), ((), ())),
        preferred_element_type=jnp.float32,
    )
    o_ref[...] = jnp.maximum(acc + b_ref[...], 0.0)


def _encode(x_bf16, W_enc_bf16, b_enc, *, br=1024, bd=2048):
    B, D_IN = x_bf16.shape
    D_DICT = W_enc_bf16.shape[0]
    grid = (D_DICT // bd, B // br)
    return pl.pallas_call(
        _enc_body,
        grid=grid,
        in_specs=[
            pl.BlockSpec((br, D_IN), lambda j, i: (i, 0)),
            pl.BlockSpec((bd, D_IN), lambda j, i: (j, 0)),
            pl.BlockSpec((1, bd), lambda j, i: (0, j)),
        ],
        out_specs=pl.BlockSpec((br, bd), lambda j, i: (i, j)),
        out_shape=jax.ShapeDtypeStruct((B, D_DICT), jnp.float32),
    )(x_bf16, W_enc_bf16, b_enc.reshape(1, D_DICT))


_J_MAX = 4


def _lane_count(m):
    w = m.shape[1]
    while w > 128:
        w //= 2
        m = jax.lax.slice(m, (0, 0), (m.shape[0], w)) + jax.lax.slice(
            m, (0, w), (m.shape[0], 2 * w))
    return jnp.sum(m.astype(jnp.float32), axis=1, keepdims=True)


def _topk_body(pre_ref, lat_ref, p16_ref):
    pre = pre_ref[...]
    bits = pltpu.bitcast(pre, jnp.int32)
    rows, _ = pre.shape

    p16_ref[...] = pltpu.bitcast(
        jnp.bitwise_and(bits, jnp.int32(-65536)), jnp.float32
    ).astype(jnp.bfloat16)

    def body16(_, carry):
        lo, hi, c_lo, c_hi = carry
        mid = lo + jax.lax.shift_right_logical(hi - lo, 1)
        t = pltpu.bitcast(mid << 16, jnp.float32).astype(jnp.bfloat16)
        mask = jnp.where(p16_ref[...] >= t, jnp.bfloat16(1), jnp.bfloat16(0))
        cnt = _lane_count(mask)
        ok = cnt >= float(_K)
        return (jnp.where(ok, mid, lo), jnp.where(ok, hi, mid),
                jnp.where(ok, cnt, c_lo), jnp.where(ok, c_hi, cnt))

    carry = (jnp.zeros((rows, 1), jnp.int32),
             jnp.full((rows, 1), 0x7F80, jnp.int32),
             jnp.full((rows, 1), float(pre.shape[1]), jnp.float32),
             jnp.zeros((rows, 1), jnp.float32))
    lo16, _, c_lo, c_hi = jax.lax.fori_loop(0, 15, body16, carry)

    def body32(_, carry):
        lo, hi, c_lo, c_hi = carry
        mid = lo + jax.lax.shift_right_logical(hi - lo, 1)
        t = pltpu.bitcast(mid, jnp.float32)
        cnt = _lane_count(jnp.where(pre >= t, 1.0, 0.0))
        ok = cnt >= float(_K)
        return (jnp.where(ok, mid, lo), jnp.where(ok, hi, mid),
                jnp.where(ok, cnt, c_lo), jnp.where(ok, c_hi, cnt))

    lo = lo16 << 16
    carry = (lo, lo + 0x10000, c_lo, c_hi)
    lo, hi, c_lo, c_hi = jax.lax.fori_loop(0, 4, body32, carry)

    bucket_lo = pltpu.bitcast(lo, jnp.float32)
    bucket_hi = pltpu.bitcast(hi, jnp.float32)
    r = float(_K) - c_hi

    z0 = jnp.where(pre < bucket_hi, pre, -1.0)
    m_j = jnp.max(z0, axis=1, keepdims=True)
    sel = jnp.where(r == 1.0, m_j, -1.0)
    for j in range(2, _J_MAX + 1):
        m_j = jnp.max(jnp.where(z0 < m_j, z0, -1.0), axis=1, keepdims=True)
        sel = jnp.where(r == float(j), m_j, sel)
    t_val = jnp.where(c_lo <= float(_K), bucket_lo,
                      jnp.maximum(sel, bucket_lo))
    lat_ref[...] = jnp.where(pre >= t_val, pre, 0.0)


def _topk_mask(pre, *, br=128):
    B, D_DICT = pre.shape
    return pl.pallas_call(
        _topk_body,
        grid=(B // br,),
        in_specs=[pl.BlockSpec((br, D_DICT), lambda i: (i, 0))],
        out_specs=pl.BlockSpec((br, D_DICT), lambda i: (i, 0)),
        out_shape=jax.ShapeDtypeStruct((B, D_DICT), jnp.float32),
        scratch_shapes=[pltpu.VMEM((br, D_DICT), jnp.bfloat16)],
    )(pre)


def _dec_body(lat_ref, w_ref, b_ref, o_ref):
    k = pl.program_id(1)

    @pl.when(k == 0)
    def _():
        o_ref[...] = jnp.broadcast_to(b_ref[...], o_ref.shape)

    lat = lat_ref[...].astype(jnp.bfloat16)
    o_ref[...] += jax.lax.dot_general(
        lat, w_ref[...],
        (((1,), (1,)), ((), ())),
        preferred_element_type=jnp.float32,
    )


def _decode(latents, W_dec_bf16, b_dec, *, br=1024, bk=2048):
    B, D_DICT = latents.shape
    D_IN = W_dec_bf16.shape[0]
    grid = (B // br, D_DICT // bk)
    return pl.pallas_call(
        _dec_body,
        grid=grid,
        in_specs=[
            pl.BlockSpec((br, bk), lambda i, k: (i, k)),
            pl.BlockSpec((D_IN, bk), lambda i, k: (0, k)),
            pl.BlockSpec((1, D_IN), lambda i, k: (0, 0)),
        ],
        out_specs=pl.BlockSpec((br, D_IN), lambda i, k: (i, 0)),
        out_shape=jax.ShapeDtypeStruct((B, D_IN), jnp.float32),
    )(latents, W_dec_bf16, b_dec.reshape(1, D_IN))


def kernel(x, W_enc, b_enc, W_dec, b_dec):
    pre = _encode(x.astype(jnp.bfloat16), W_enc.astype(jnp.bfloat16), b_enc)
    latents = _topk_mask(pre)
    x_hat = _decode(latents, W_dec.astype(jnp.bfloat16), b_dec)
    return (x_hat, latents)


# speedup vs baseline: 1.0489x; 1.0057x over previous
import jax
import jax.numpy as jnp
from jax.experimental import pallas as pl
from jax.experimental.pallas import tpu as pltpu

_K = 64


def _enc_body(x_ref, w_ref, b_ref, o_ref):
    acc = jax.lax.dot_general(
        x_ref[...], w_ref[...],
        (((1,), (1,)), ((), ())),
        preferred_element_type=jnp.float32,
    )
    o_ref[...] = jnp.maximum(acc + b_ref[...], 0.0)


def _encode(x_bf16, W_enc_bf16, b_enc, *, br=1024, bd=2048):
    B, D_IN = x_bf16.shape
    D_DICT = W_enc_bf16.shape[0]
    grid = (D_DICT // bd, B // br)
    return pl.pallas_call(
        _enc_body,
        grid=grid,
        in_specs=[
            pl.BlockSpec((br, D_IN), lambda j, i: (i, 0)),
            pl.BlockSpec((bd, D_IN), lambda j, i: (j, 0)),
            pl.BlockSpec((1, bd), lambda j, i: (0, j)),
        ],
        out_specs=pl.BlockSpec((br, bd), lambda j, i: (i, j)),
        out_shape=jax.ShapeDtypeStruct((B, D_DICT), jnp.float32),
    )(x_bf16, W_enc_bf16, b_enc.reshape(1, D_DICT))


_J_MAX = 4


def _lane_count(m):
    w = m.shape[1]
    while w > 128:
        w //= 2
        m = jax.lax.slice(m, (0, 0), (m.shape[0], w)) + jax.lax.slice(
            m, (0, w), (m.shape[0], 2 * w))
    return jnp.sum(m.astype(jnp.float32), axis=1, keepdims=True)


def _topk_body(pre_ref, t_ref, p16_ref):
    pre = pre_ref[...]
    bits = pltpu.bitcast(pre, jnp.int32)
    rows, _ = pre.shape

    p16_ref[...] = pltpu.bitcast(
        jnp.bitwise_and(bits, jnp.int32(-65536)), jnp.float32
    ).astype(jnp.bfloat16)

    def body16(_, carry):
        lo, hi, c_lo, c_hi = carry
        mid = lo + jax.lax.shift_right_logical(hi - lo, 1)
        t = pltpu.bitcast(mid << 16, jnp.float32).astype(jnp.bfloat16)
        mask = jnp.where(p16_ref[...] >= t, jnp.bfloat16(1), jnp.bfloat16(0))
        cnt = _lane_count(mask)
        ok = cnt >= float(_K)
        return (jnp.where(ok, mid, lo), jnp.where(ok, hi, mid),
                jnp.where(ok, cnt, c_lo), jnp.where(ok, c_hi, cnt))

    carry = (jnp.zeros((rows, 1), jnp.int32),
             jnp.full((rows, 1), 0x7F80, jnp.int32),
             jnp.full((rows, 1), float(pre.shape[1]), jnp.float32),
             jnp.zeros((rows, 1), jnp.float32))
    lo16, _, c_lo, c_hi = jax.lax.fori_loop(0, 15, body16, carry)

    def body32(_, carry):
        lo, hi, c_lo, c_hi = carry
        mid = lo + jax.lax.shift_right_logical(hi - lo, 1)
        t = pltpu.bitcast(mid, jnp.float32)
        cnt = _lane_count(jnp.where(pre >= t, 1.0, 0.0))
        ok = cnt >= float(_K)
        return (jnp.where(ok, mid, lo), jnp.where(ok, hi, mid),
                jnp.where(ok, cnt, c_lo), jnp.where(ok, c_hi, cnt))

    lo = lo16 << 16
    carry = (lo, lo + 0x10000, c_lo, c_hi)
    lo, hi, c_lo, c_hi = jax.lax.fori_loop(0, 4, body32, carry)

    bucket_lo = pltpu.bitcast(lo, jnp.float32)
    bucket_hi = pltpu.bitcast(hi, jnp.float32)
    r = float(_K) - c_hi

    z0 = jnp.where(pre < bucket_hi, pre, -1.0)
    m_j = jnp.max(z0, axis=1, keepdims=True)
    sel = jnp.where(r == 1.0, m_j, -1.0)
    for j in range(2, _J_MAX + 1):
        m_j = jnp.max(jnp.where(z0 < m_j, z0, -1.0), axis=1, keepdims=True)
        sel = jnp.where(r == float(j), m_j, sel)
    t_val = jnp.where(c_lo <= float(_K), bucket_lo,
                      jnp.maximum(sel, bucket_lo))
    t_ref[...] = jnp.broadcast_to(t_val, t_ref.shape)


def _topk_thresh(pre, *, br=256):
    B, D_DICT = pre.shape
    return pl.pallas_call(
        _topk_body,
        grid=(B // br,),
        in_specs=[pl.BlockSpec((br, D_DICT), lambda i: (i, 0))],
        out_specs=pl.BlockSpec((br, 128), lambda i: (i, 0)),
        out_shape=jax.ShapeDtypeStruct((B, 128), jnp.float32),
        scratch_shapes=[pltpu.VMEM((br, D_DICT), jnp.bfloat16)],
    )(pre)


def _dec_body(pre_ref, t_ref, w_ref, b_ref, lat_ref, o_ref):
    k = pl.program_id(1)

    @pl.when(k == 0)
    def _():
        o_ref[...] = jnp.broadcast_to(b_ref[...], o_ref.shape)

    t = jax.lax.slice(t_ref[...], (0, 0), (t_ref.shape[0], 1))
    lat = jnp.where(pre_ref[...] >= t, pre_ref[...], 0.0)
    lat_ref[...] = lat
    o_ref[...] += jax.lax.dot_general(
        lat.astype(jnp.bfloat16), w_ref[...],
        (((1,), (1,)), ((), ())),
        preferred_element_type=jnp.float32,
    )


def _mask_decode(pre, thresh, W_dec_bf16, b_dec, *, br=1024, bk=1024):
    B, D_DICT = pre.shape
    D_IN = W_dec_bf16.shape[0]
    grid = (B // br, D_DICT // bk)
    return pl.pallas_call(
        _dec_body,
        grid=grid,
        in_specs=[
            pl.BlockSpec((br, bk), lambda i, k: (i, k)),
            pl.BlockSpec((br, 128), lambda i, k: (i, 0)),
            pl.BlockSpec((D_IN, bk), lambda i, k: (0, k)),
            pl.BlockSpec((1, D_IN), lambda i, k: (0, 0)),
        ],
        out_specs=[
            pl.BlockSpec((br, bk), lambda i, k: (i, k)),
            pl.BlockSpec((br, D_IN), lambda i, k: (i, 0)),
        ],
        out_shape=[
            jax.ShapeDtypeStruct((B, D_DICT), jnp.float32),
            jax.ShapeDtypeStruct((B, D_IN), jnp.float32),
        ],
    )(pre, thresh, W_dec_bf16, b_dec.reshape(1, D_IN))


def kernel(x, W_enc, b_enc, W_dec, b_dec):
    pre = _encode(x.astype(jnp.bfloat16), W_enc.astype(jnp.bfloat16), b_enc)
    thresh = _topk_thresh(pre)
    latents, x_hat = _mask_decode(pre, thresh, W_dec.astype(jnp.bfloat16), b_dec)
    return (x_hat, latents)
